# Initial kernel scaffold; baseline (speedup 1.0000x reference)
import jax
import jax.numpy as jnp
from jax.experimental import pallas as pl


def kernel(x, codebook):
    raise NotImplementedError("write your pallas kernel here")


# speedup vs baseline: 1.2018x; 1.2018x over previous
import functools

import jax
import jax.numpy as jnp
from jax.experimental import pallas as pl

NUM_CODES = 1024
CODE_DIM = 64
BETA = 0.25


def _vq_kernel(x_ref, cb_ref, cbt_ref, zq_ref, idx_ref, loss_ref, *, nb, n_elems):
    b = pl.program_id(0)
    xm = x_ref[0]
    cb = cb_ref[...]
    cbt = cbt_ref[...]

    zc = jax.lax.dot_general(
        cb, xm, (((1,), (0,)), ((), ())),
        preferred_element_type=jnp.float32,
        precision=jax.lax.Precision.DEFAULT,
    )
    c2 = jnp.sum(cb * cb, axis=1, keepdims=True)
    z2 = jnp.sum(xm * xm, axis=0, keepdims=True)
    dist = z2 + c2 - 2.0 * zc

    minv = jnp.min(dist, axis=0, keepdims=True)
    code_iota = jax.lax.broadcasted_iota(jnp.int32, dist.shape, 0)
    idx = jnp.min(jnp.where(dist == minv, code_iota, NUM_CODES), axis=0)
    idx_ref[0, 0, :] = idx

    onehot = (code_iota == idx[None, :]).astype(jnp.float32)
    zq = jax.lax.dot_general(
        cbt, onehot, (((1,), (0,)), ((), ())),
        preferred_element_type=jnp.float32,
        precision=jax.lax.Precision.HIGHEST,
    )
    zq_ref[0] = zq

    diff = zq - xm
    part = jnp.sum(diff * diff).reshape(1, 1)
    total = jnp.where(b == 0, part, loss_ref[...] + part)
    scale = (1.0 + BETA) / n_elems
    loss_ref[...] = jnp.where(b == nb - 1, total * scale, total)


def kernel(x, codebook):
    B, D, H, W = x.shape
    P = H * W
    x3 = x.reshape(B, D, P)
    cbt = codebook.T

    zq3, idx3, loss = pl.pallas_call(
        functools.partial(_vq_kernel, nb=B, n_elems=x.size),
        grid=(B,),
        in_specs=[
            pl.BlockSpec((1, D, P), lambda b: (b, 0, 0)),
            pl.BlockSpec((NUM_CODES, CODE_DIM), lambda b: (0, 0)),
            pl.BlockSpec((CODE_DIM, NUM_CODES), lambda b: (0, 0)),
        ],
        out_specs=[
            pl.BlockSpec((1, D, P), lambda b: (b, 0, 0)),
            pl.BlockSpec((1, 1, P), lambda b: (b, 0, 0)),
            pl.BlockSpec((1, 1), lambda b: (0, 0)),
        ],
        out_shape=[
            jax.ShapeDtypeStruct((B, D, P), jnp.float32),
            jax.ShapeDtypeStruct((B, 1, P), jnp.int32),
            jax.ShapeDtypeStruct((1, 1), jnp.float32),
        ],
    )(x3, codebook, cbt)

    z_q = zq3.reshape(B, D, H, W)
    encoding_indices = idx3.reshape(B * P)
    vq_loss = loss[0, 0]
    return (z_q, vq_loss, encoding_indices)


# speedup vs baseline: 1.9269x; 1.6033x over previous
import functools

import jax
import jax.numpy as jnp
from jax.experimental import pallas as pl

NUM_CODES = 1024
CODE_DIM = 64
BETA = 0.25


def _vq_kernel(x_ref, cb_ref, cbth_ref, cbtl_ref, zq_ref, idx_ref, loss_ref,
               *, nb, n_elems):
    b = pl.program_id(0)
    xm = x_ref[0]
    cb = cb_ref[...]

    zc = jax.lax.dot_general(
        cb, xm, (((1,), (0,)), ((), ())),
        preferred_element_type=jnp.float32,
        precision=jax.lax.Precision.DEFAULT,
    )
    c2 = jnp.sum(cb * cb, axis=1, keepdims=True)
    z2 = jnp.sum(xm * xm, axis=0, keepdims=True)
    dist = z2 + c2 - 2.0 * zc

    minv = jnp.min(dist, axis=0, keepdims=True)
    code_iota = jax.lax.broadcasted_iota(jnp.int32, dist.shape, 0)
    idx = jnp.min(jnp.where(dist == minv, code_iota, NUM_CODES), axis=0)
    idx_ref[0, 0, :] = idx

    onehot = (code_iota == idx[None, :]).astype(jnp.bfloat16)
    dn = (((1,), (0,)), ((), ()))
    zq = (jax.lax.dot_general(cbth_ref[...], onehot, dn,
                              preferred_element_type=jnp.float32)
          + jax.lax.dot_general(cbtl_ref[...], onehot, dn,
                                preferred_element_type=jnp.float32))
    zq_ref[0] = zq

    part = jnp.sum(minv).reshape(1, 1)
    total = jnp.where(b == 0, part, loss_ref[...] + part)
    scale = (1.0 + BETA) / n_elems
    loss_ref[...] = jnp.where(b == nb - 1, total * scale, total)


def kernel(x, codebook):
    B, D, H, W = x.shape
    P = H * W
    x3 = x.reshape(B, D, P)
    cbt = codebook.T
    cbt_hi = cbt.astype(jnp.bfloat16)
    cbt_lo = (cbt - cbt_hi.astype(jnp.float32)).astype(jnp.bfloat16)

    zq3, idx3, loss = pl.pallas_call(
        functools.partial(_vq_kernel, nb=B, n_elems=x.size),
        grid=(B,),
        in_specs=[
            pl.BlockSpec((1, D, P), lambda b: (b, 0, 0)),
            pl.BlockSpec((NUM_CODES, CODE_DIM), lambda b: (0, 0)),
            pl.BlockSpec((CODE_DIM, NUM_CODES), lambda b: (0, 0)),
            pl.BlockSpec((CODE_DIM, NUM_CODES), lambda b: (0, 0)),
        ],
        out_specs=[
            pl.BlockSpec((1, D, P), lambda b: (b, 0, 0)),
            pl.BlockSpec((1, 1, P), lambda b: (b, 0, 0)),
            pl.BlockSpec((1, 1), lambda b: (0, 0)),
        ],
        out_shape=[
            jax.ShapeDtypeStruct((B, D, P), jnp.float32),
            jax.ShapeDtypeStruct((B, 1, P), jnp.int32),
            jax.ShapeDtypeStruct((1, 1), jnp.float32),
        ],
    )(x3, codebook, cbt_hi, cbt_lo)

    z_q = zq3.reshape(B, D, H, W)
    encoding_indices = idx3.reshape(B * P)
    vq_loss = loss[0, 0]
    return (z_q, vq_loss, encoding_indices)
